# baseline (speedup 1.0000x reference)
import functools

import jax
import jax.numpy as jnp
from jax import lax
from jax.experimental import pallas as pl
from jax.experimental.pallas import tpu as pltpu
from jax.experimental.pallas import tpu_sc as plsc

D = 128
NROWS = 3120
FIDX_BLOCK = 2048


def _table_body(t_ref, out_ref):
    i2 = lax.broadcasted_iota(jnp.int32, (NROWS, 32), 0)
    l2 = lax.broadcasted_iota(jnp.int32, (NROWS, 32), 1)
    c = i2 // 60
    v = i2 - c * 60
    q = c // 13
    r = c - q * 13
    st = v // 15
    rem = v - st * 15
    vi = rem // 5
    o = rem - vi * 5
    oh = (
        (l2 == r)
        | (l2 == 13 + q)
        | (l2 == 17 + st)
        | (l2 == 21 + vi)
        | (l2 == 24 + o)
    ).astype(jnp.float32)
    out_ref[...] = jnp.dot(oh, t_ref[...], preferred_element_type=jnp.float32)


def _fidx_body(c_ref, st_ref, vi_ref, o_ref, out_ref):
    out_ref[...] = (
        c_ref[...] * 60 + st_ref[...] * 15 + vi_ref[...] * 5 + o_ref[...]
    )


def _make_sc_kernel(n_rows_out):
    info = plsc.get_sparse_core_info()
    nc, ns = info.num_cores, info.num_subcores
    nw = nc * ns
    idx_rows = n_rows_out // D
    per_w = idx_rows // nw
    nbuf = 3
    ci = 2
    n_chunks = per_w // ci

    mesh = plsc.VectorSubcoreMesh(core_axis_name="c", subcore_axis_name="s")

    @functools.partial(
        pl.kernel,
        mesh=mesh,
        out_type=jax.ShapeDtypeStruct((n_rows_out, D), jnp.float32),
        scratch_types=[
            pltpu.VMEM((nbuf, ci, D), jnp.int32),
            pltpu.VMEM((nbuf, ci * D, D), jnp.float32),
            pltpu.VMEM_SHARED((NROWS, D), jnp.float32),
            pltpu.SemaphoreType.DMA((nbuf,)),
            pltpu.SemaphoreType.DMA((nbuf,)),
            pltpu.SemaphoreType.DMA((nbuf,)),
        ],
    )
    def sc_gather(table_hbm, fidx_hbm, out_hbm, idx_v, rows_v, table_sh,
                  sem_i, sem_g, sem_o):
        wid = lax.axis_index("s") * nc + lax.axis_index("c")
        ibase = wid * per_w

        sid = lax.axis_index("s")
        pltpu.sync_copy(
            table_hbm.at[pl.ds(sid * 192, 192)],
            table_sh.at[pl.ds(sid * 192, 192)])

        @pl.when(sid == ns - 1)
        def _():
            pltpu.sync_copy(
                table_hbm.at[pl.ds(ns * 192, NROWS - ns * 192)],
                table_sh.at[pl.ds(ns * 192, NROWS - ns * 192)])

        plsc.subcore_barrier()

        def fetch_idx(g, b):
            pltpu.async_copy(
                fidx_hbm.at[pl.ds(ibase + g * ci, ci)], idx_v.at[b], sem_i.at[b])

        def wait_idx(b):
            pltpu.make_async_copy(
                fidx_hbm.at[pl.ds(ibase, ci)], idx_v.at[b], sem_i.at[b]).wait()

        def fire_gather(b):
            for j in range(ci):
                pltpu.async_copy(
                    table_sh.at[idx_v.at[b, j]],
                    rows_v.at[b, pl.ds(j * D, D)], sem_g.at[b])

        def drain_gather(b):
            for j in range(ci):
                pltpu.make_async_copy(
                    table_sh.at[idx_v.at[b, j]],
                    rows_v.at[b, pl.ds(j * D, D)], sem_g.at[b]).wait()

        def store_out(g, b):
            pltpu.async_copy(
                rows_v.at[b],
                out_hbm.at[pl.ds(ibase * D + g * ci * D, ci * D)], sem_o.at[b])

        def wait_store(b):
            pltpu.make_async_copy(
                rows_v.at[b],
                out_hbm.at[pl.ds(ibase * D, ci * D)], sem_o.at[b]).wait()

        for b in range(nbuf):
            fetch_idx(b, b)
        wait_idx(0)
        fire_gather(0)
        wait_idx(1)
        fire_gather(1)

        def body(g4, _):
            for b in range(nbuf):
                g = g4 * nbuf + b
                drain_gather(b)
                store_out(g, b)

                @pl.when(g >= nbuf - 2)
                def _():
                    wait_store((b + 2) % nbuf)

                @pl.when(g + 2 < n_chunks)
                def _():
                    wait_idx((b + 2) % nbuf)
                    fire_gather((b + 2) % nbuf)

                @pl.when(g + nbuf < n_chunks)
                def _():
                    fetch_idx(g + nbuf, b)
            return 0

        n_full = n_chunks // nbuf
        rem = n_chunks - n_full * nbuf
        assert rem <= 2
        lax.fori_loop(0, n_full, body, 0)
        for r in range(rem):
            g = n_full * nbuf + r
            b = g % nbuf
            drain_gather(b)
            store_out(g, b)
            if g >= nbuf - 2:
                wait_store((b + 2) % nbuf)
        for t in range(nbuf - 2, 0, -1):
            wait_store((n_chunks - t) % nbuf)

    return sc_gather


def kernel(card_indices, stages, visibility, order, rank_emb, suit_emb,
           stage_emb, visibility_emb, order_emb):
    B, L = card_indices.shape
    N = B * L
    tables = jnp.concatenate(
        [rank_emb, suit_emb, stage_emb, visibility_emb, order_emb,
         jnp.zeros((3, D), jnp.float32)], axis=0)

    fused_table = pl.pallas_call(
        _table_body,
        in_specs=[pl.BlockSpec((32, D), lambda: (0, 0))],
        out_specs=pl.BlockSpec((NROWS, D), lambda: (0, 0)),
        out_shape=jax.ShapeDtypeStruct((NROWS, D), jnp.float32),
    )(tables)

    nb = B // FIDX_BLOCK
    spec = pl.BlockSpec((L, FIDX_BLOCK), lambda i: (0, i))
    fidx_t = pl.pallas_call(
        _fidx_body,
        grid=(nb,),
        in_specs=[spec, spec, spec, spec],
        out_specs=spec,
        out_shape=jax.ShapeDtypeStruct((L, B), jnp.int32),
    )(card_indices.T.astype(jnp.int32), stages.T.astype(jnp.int32),
      visibility.T.astype(jnp.int32), order.T.astype(jnp.int32))

    fidx2d = fidx_t.reshape(N // D, D)
    out = _make_sc_kernel(N)(fused_table, fidx2d)
    return out.reshape(L, B, D).transpose(1, 0, 2)


# speedup vs baseline: 1.0474x; 1.0474x over previous
import functools

import jax
import jax.numpy as jnp
from jax import lax
from jax.experimental import pallas as pl
from jax.experimental.pallas import tpu as pltpu
from jax.experimental.pallas import tpu_sc as plsc

D = 128
NROWS = 3120
FIDX_BLOCK = 2048


def _prep_body(t_ref, c_ref, st_ref, vi_ref, o_ref, fidx_ref, tout_ref):
    @pl.when(pl.program_id(0) == 0)
    def _():
        _build_table(t_ref, tout_ref)

    fidx_ref[...] = (
        c_ref[...] * 60 + st_ref[...] * 15 + vi_ref[...] * 5 + o_ref[...]
    )


def _build_table(t_ref, out_ref):
    i2 = lax.broadcasted_iota(jnp.int32, (NROWS, 32), 0)
    l2 = lax.broadcasted_iota(jnp.int32, (NROWS, 32), 1)
    c = i2 // 60
    v = i2 - c * 60
    q = c // 13
    r = c - q * 13
    st = v // 15
    rem = v - st * 15
    vi = rem // 5
    o = rem - vi * 5
    oh = (
        (l2 == r)
        | (l2 == 13 + q)
        | (l2 == 17 + st)
        | (l2 == 21 + vi)
        | (l2 == 24 + o)
    ).astype(jnp.float32)
    out_ref[...] = jnp.dot(oh, t_ref[...], preferred_element_type=jnp.float32)


def _make_sc_kernel(n_rows_out):
    info = plsc.get_sparse_core_info()
    nc, ns = info.num_cores, info.num_subcores
    nw = nc * ns
    idx_rows = n_rows_out // D
    per_w = idx_rows // nw
    nbuf = 4
    n_chunks = per_w

    mesh = plsc.VectorSubcoreMesh(core_axis_name="c", subcore_axis_name="s")

    @functools.partial(
        pl.kernel,
        mesh=mesh,
        out_type=jax.ShapeDtypeStruct((n_rows_out, D), jnp.float32),
        scratch_types=[
            pltpu.VMEM((nbuf, 1, D), jnp.int32),
            pltpu.VMEM((nbuf, D, D), jnp.float32),
            pltpu.VMEM_SHARED((NROWS, D), jnp.float32),
            pltpu.SemaphoreType.DMA((nbuf,)),
            pltpu.SemaphoreType.DMA((nbuf,)),
            pltpu.SemaphoreType.DMA((nbuf,)),
        ],
    )
    def sc_gather(table_hbm, fidx_hbm, out_hbm, idx_v, rows_v, table_sh,
                  sem_i, sem_g, sem_o):
        wid = lax.axis_index("s") * nc + lax.axis_index("c")
        ibase = wid * per_w

        sid = lax.axis_index("s")
        pltpu.sync_copy(
            table_hbm.at[pl.ds(sid * 192, 192)],
            table_sh.at[pl.ds(sid * 192, 192)])

        @pl.when(sid == ns - 1)
        def _():
            pltpu.sync_copy(
                table_hbm.at[pl.ds(ns * 192, NROWS - ns * 192)],
                table_sh.at[pl.ds(ns * 192, NROWS - ns * 192)])

        plsc.subcore_barrier()

        def fetch_idx(g, b):
            pltpu.async_copy(
                fidx_hbm.at[pl.ds(ibase + g, 1)], idx_v.at[b], sem_i.at[b])

        def wait_idx(b):
            pltpu.make_async_copy(
                fidx_hbm.at[pl.ds(ibase, 1)], idx_v.at[b], sem_i.at[b]).wait()

        def fire_gather(b):
            pltpu.async_copy(
                table_sh.at[idx_v.at[b, 0]], rows_v.at[b], sem_g.at[b])

        def drain_gather(b):
            pltpu.make_async_copy(
                table_sh.at[idx_v.at[b, 0]], rows_v.at[b], sem_g.at[b]).wait()

        def store_out(g, b):
            pltpu.async_copy(
                rows_v.at[b],
                out_hbm.at[pl.ds(ibase * D + g * D, D)], sem_o.at[b])

        def wait_store(b):
            pltpu.make_async_copy(
                rows_v.at[b],
                out_hbm.at[pl.ds(ibase * D, D)], sem_o.at[b]).wait()

        for b in range(nbuf):
            fetch_idx(b, b)
        wait_idx(0)
        fire_gather(0)
        wait_idx(1)
        fire_gather(1)

        def body(g4, _):
            for b in range(nbuf):
                g = g4 * nbuf + b
                drain_gather(b)
                store_out(g, b)

                @pl.when(g >= 2)
                def _():
                    wait_store((b + 2) % nbuf)

                @pl.when(g + 2 < n_chunks)
                def _():
                    wait_idx((b + 2) % nbuf)
                    fire_gather((b + 2) % nbuf)

                @pl.when(g + nbuf < n_chunks)
                def _():
                    fetch_idx(g + nbuf, b)
            return 0

        lax.fori_loop(0, n_chunks // nbuf, body, 0)
        wait_store((n_chunks - 2) % nbuf)
        wait_store((n_chunks - 1) % nbuf)

    return sc_gather


def kernel(card_indices, stages, visibility, order, rank_emb, suit_emb,
           stage_emb, visibility_emb, order_emb):
    B, L = card_indices.shape
    N = B * L
    tables = jnp.concatenate(
        [rank_emb, suit_emb, stage_emb, visibility_emb, order_emb,
         jnp.zeros((3, D), jnp.float32)], axis=0)

    nb = B // FIDX_BLOCK
    spec = pl.BlockSpec((L, FIDX_BLOCK), lambda i: (0, i))
    fidx_t, fused_table = pl.pallas_call(
        _prep_body,
        grid=(nb,),
        in_specs=[pl.BlockSpec((32, D), lambda i: (0, 0)),
                  spec, spec, spec, spec],
        out_specs=[spec, pl.BlockSpec((NROWS, D), lambda i: (0, 0))],
        out_shape=[jax.ShapeDtypeStruct((L, B), jnp.int32),
                   jax.ShapeDtypeStruct((NROWS, D), jnp.float32)],
    )(tables, card_indices.T.astype(jnp.int32), stages.T.astype(jnp.int32),
      visibility.T.astype(jnp.int32), order.T.astype(jnp.int32))

    fidx2d = fidx_t.reshape(N // D, D)
    out = _make_sc_kernel(N)(fused_table, fidx2d)
    return out.reshape(L, B, D).transpose(1, 0, 2)


# speedup vs baseline: 1.0554x; 1.0076x over previous
import functools

import jax
import jax.numpy as jnp
from jax import lax
from jax.experimental import pallas as pl
from jax.experimental.pallas import tpu as pltpu
from jax.experimental.pallas import tpu_sc as plsc

D = 128
NROWS = 3120
FIDX_BLOCK = 4096


def _prep_body(t_ref, c_ref, st_ref, vi_ref, o_ref, fidx_ref, tout_ref):
    @pl.when(pl.program_id(0) == 0)
    def _():
        _build_table(t_ref, tout_ref)

    fidx_ref[...] = (
        c_ref[...] * 60 + st_ref[...] * 15 + vi_ref[...] * 5 + o_ref[...]
    )


def _build_table(t_ref, out_ref):
    i2 = lax.broadcasted_iota(jnp.int32, (NROWS, 32), 0)
    l2 = lax.broadcasted_iota(jnp.int32, (NROWS, 32), 1)
    c = i2 // 60
    v = i2 - c * 60
    q = c // 13
    r = c - q * 13
    st = v // 15
    rem = v - st * 15
    vi = rem // 5
    o = rem - vi * 5
    oh = (
        (l2 == r)
        | (l2 == 13 + q)
        | (l2 == 17 + st)
        | (l2 == 21 + vi)
        | (l2 == 24 + o)
    ).astype(jnp.float32)
    out_ref[...] = jnp.dot(oh, t_ref[...], preferred_element_type=jnp.float32)


def _make_sc_kernel(n_rows_out):
    info = plsc.get_sparse_core_info()
    nc, ns = info.num_cores, info.num_subcores
    nw = nc * ns
    idx_rows = n_rows_out // D
    per_w = idx_rows // nw
    nbuf = 4
    n_chunks = per_w

    mesh = plsc.VectorSubcoreMesh(core_axis_name="c", subcore_axis_name="s")

    @functools.partial(
        pl.kernel,
        mesh=mesh,
        out_type=jax.ShapeDtypeStruct((n_rows_out, D), jnp.float32),
        scratch_types=[
            pltpu.VMEM((nbuf, 1, D), jnp.int32),
            pltpu.VMEM((nbuf, D, D), jnp.float32),
            pltpu.VMEM_SHARED((NROWS, D), jnp.float32),
            pltpu.SemaphoreType.DMA((nbuf,)),
            pltpu.SemaphoreType.DMA((nbuf,)),
            pltpu.SemaphoreType.DMA((nbuf,)),
        ],
    )
    def sc_gather(table_hbm, fidx_hbm, out_hbm, idx_v, rows_v, table_sh,
                  sem_i, sem_g, sem_o):
        wid = lax.axis_index("s") * nc + lax.axis_index("c")
        ibase = wid * per_w

        sid = lax.axis_index("s")
        pltpu.sync_copy(
            table_hbm.at[pl.ds(sid * 192, 192)],
            table_sh.at[pl.ds(sid * 192, 192)])

        @pl.when(sid == ns - 1)
        def _():
            pltpu.sync_copy(
                table_hbm.at[pl.ds(ns * 192, NROWS - ns * 192)],
                table_sh.at[pl.ds(ns * 192, NROWS - ns * 192)])

        plsc.subcore_barrier()

        def fetch_idx(g, b):
            pltpu.async_copy(
                fidx_hbm.at[pl.ds(ibase + g, 1)], idx_v.at[b], sem_i.at[b])

        def wait_idx(b):
            pltpu.make_async_copy(
                fidx_hbm.at[pl.ds(ibase, 1)], idx_v.at[b], sem_i.at[b]).wait()

        def fire_gather(b):
            pltpu.async_copy(
                table_sh.at[idx_v.at[b, 0]], rows_v.at[b], sem_g.at[b])

        def drain_gather(b):
            pltpu.make_async_copy(
                table_sh.at[idx_v.at[b, 0]], rows_v.at[b], sem_g.at[b]).wait()

        def store_out(g, b):
            pltpu.async_copy(
                rows_v.at[b],
                out_hbm.at[pl.ds(ibase * D + g * D, D)], sem_o.at[b])

        def wait_store(b):
            pltpu.make_async_copy(
                rows_v.at[b],
                out_hbm.at[pl.ds(ibase * D, D)], sem_o.at[b]).wait()

        for b in range(nbuf):
            fetch_idx(b, b)
        wait_idx(0)
        fire_gather(0)
        wait_idx(1)
        fire_gather(1)

        def body(g4, _):
            for b in range(nbuf):
                g = g4 * nbuf + b
                drain_gather(b)
                store_out(g, b)

                @pl.when(g >= 2)
                def _():
                    wait_store((b + 2) % nbuf)

                @pl.when(g + 2 < n_chunks)
                def _():
                    wait_idx((b + 2) % nbuf)
                    fire_gather((b + 2) % nbuf)

                @pl.when(g + nbuf < n_chunks)
                def _():
                    fetch_idx(g + nbuf, b)
            return 0

        lax.fori_loop(0, n_chunks // nbuf, body, 0)
        wait_store((n_chunks - 2) % nbuf)
        wait_store((n_chunks - 1) % nbuf)

    return sc_gather


def kernel(card_indices, stages, visibility, order, rank_emb, suit_emb,
           stage_emb, visibility_emb, order_emb):
    B, L = card_indices.shape
    N = B * L
    tables = jnp.concatenate(
        [rank_emb, suit_emb, stage_emb, visibility_emb, order_emb,
         jnp.zeros((3, D), jnp.float32)], axis=0)

    nb = B // FIDX_BLOCK
    spec = pl.BlockSpec((L, FIDX_BLOCK), lambda i: (0, i))
    fidx_t, fused_table = pl.pallas_call(
        _prep_body,
        grid=(nb,),
        in_specs=[pl.BlockSpec((32, D), lambda i: (0, 0)),
                  spec, spec, spec, spec],
        out_specs=[spec, pl.BlockSpec((NROWS, D), lambda i: (0, 0))],
        out_shape=[jax.ShapeDtypeStruct((L, B), jnp.int32),
                   jax.ShapeDtypeStruct((NROWS, D), jnp.float32)],
    )(tables, card_indices.T.astype(jnp.int32), stages.T.astype(jnp.int32),
      visibility.T.astype(jnp.int32), order.T.astype(jnp.int32))

    fidx2d = fidx_t.reshape(N // D, D)
    out = _make_sc_kernel(N)(fused_table, fidx2d)
    return out.reshape(L, B, D).transpose(1, 0, 2)
